# baseline (speedup 1.0000x reference)
import functools

import jax
import jax.numpy as jnp
from jax import lax
from jax.experimental import pallas as pl
from jax.experimental.pallas import tpu as pltpu
from jax.experimental.pallas import tpu_sc as plsc

N = 10000
E = 320000
D = 128
NC, NS = 2, 16
NW = NC * NS
EPW = E // NW
C = 80
NP = 10240
RPT = NP // NS
HR = 128
CB = 2000
BN = 2000


def _pre_body(h_ref, cr_ref, w1_ref, b1_ref, ee_ref, v_ref):
    h = h_ref[...]
    p = jnp.dot(h, w1_ref[0:128, :], preferred_element_type=jnp.float32,
                precision=lax.Precision.HIGHEST) + cr_ref[...] * w1_ref[320:321, :]
    q = jnp.dot(h, w1_ref[128:256, :], preferred_element_type=jnp.float32,
                precision=lax.Precision.HIGHEST)
    tb = jnp.dot(ee_ref[...], w1_ref[256:320, :],
                 preferred_element_type=jnp.float32,
                 precision=lax.Precision.HIGHEST) + b1_ref[...]
    parts = [(p + tb[t:t + 1, :])[:, None, :] for t in range(4)]
    parts.append(q[:, None, :])
    v_ref[...] = jnp.concatenate(parts, axis=1)


def _tc_pre(h, cr2, W1, b12, ee):
    return pl.pallas_call(
        _pre_body,
        grid=(N // BN,),
        in_specs=[
            pl.BlockSpec((BN, D), lambda i: (i, 0)),
            pl.BlockSpec((BN, 1), lambda i: (i, 0)),
            pl.BlockSpec((321, D), lambda i: (0, 0)),
            pl.BlockSpec((1, D), lambda i: (0, 0)),
            pl.BlockSpec((4, 64), lambda i: (0, 0)),
        ],
        out_specs=pl.BlockSpec((BN, 5, D), lambda i: (i, 0, 0)),
        out_shape=jax.ShapeDtypeStruct((N, 5, D), jnp.float32),
    )(h, cr2, W1, b12, ee)


def _post_body(s_ref, c_ref, h_ref, w2_ref, b2_ref, wih_ref, bih_ref,
               whh_ref, bhh_ref, out_ref):
    s = s_ref[0] + s_ref[1]
    cnt = jnp.sum(c_ref[...], axis=0)
    h = h_ref[...]
    denom = jnp.maximum(cnt, 1.0)
    sw = jnp.dot(s, w2_ref[...], preferred_element_type=jnp.float32,
                 precision=lax.Precision.HIGHEST)
    agg = (sw + cnt * b2_ref[...]) / denom
    gi = jnp.dot(agg, wih_ref[...], preferred_element_type=jnp.float32,
                 precision=lax.Precision.HIGHEST) + bih_ref[...]
    gh = jnp.dot(h, whh_ref[...], preferred_element_type=jnp.float32,
                 precision=lax.Precision.HIGHEST) + bhh_ref[...]
    rg = jax.nn.sigmoid(gi[:, 0:128] + gh[:, 0:128])
    zg = jax.nn.sigmoid(gi[:, 128:256] + gh[:, 128:256])
    ng = jnp.tanh(gi[:, 256:384] + rg * gh[:, 256:384])
    h_new = (1.0 - zg) * ng + zg * h
    out_ref[...] = jnp.where(cnt > 0, h_new, h)


def _tc_post(sacc, cnt2, h, W2, b22, Wih, bih2, Whh, bhh2):
    return pl.pallas_call(
        _post_body,
        grid=(N // BN,),
        in_specs=[
            pl.BlockSpec((NC, BN, D), lambda i: (0, i, 0)),
            pl.BlockSpec((NC, BN, 1), lambda i: (0, i, 0)),
            pl.BlockSpec((BN, D), lambda i: (i, 0)),
            pl.BlockSpec((D, D), lambda i: (0, 0)),
            pl.BlockSpec((1, D), lambda i: (0, 0)),
            pl.BlockSpec((D, 3 * D), lambda i: (0, 0)),
            pl.BlockSpec((1, 3 * D), lambda i: (0, 0)),
            pl.BlockSpec((D, 3 * D), lambda i: (0, 0)),
            pl.BlockSpec((1, 3 * D), lambda i: (0, 0)),
        ],
        out_specs=pl.BlockSpec((BN, D), lambda i: (i, 0)),
        out_shape=jax.ShapeDtypeStruct((N, D), jnp.float32),
    )(sacc, cnt2, h, W2, b22, Wih, bih2, Whh, bhh2)


def _build_sc_edges():
    mesh = plsc.VectorSubcoreMesh(core_axis_name="c", subcore_axis_name="s")
    NCH = EPW // C
    SG = 16
    NS_G = C // SG

    @functools.partial(
        pl.kernel,
        mesh=mesh,
        out_type=jax.ShapeDtypeStruct((NC, NP, D), jnp.float32),
        scratch_types=[
            pltpu.VMEM((2, 1, C), jnp.int32),
            pltpu.VMEM((2, 4, C), jnp.int32),
            pltpu.VMEM((2, C, D), jnp.float32),
            pltpu.VMEM((2, C, D), jnp.float32),
            pltpu.SemaphoreType.DMA((2,)),
            pltpu.SemaphoreType.DMA((2,)),
            pltpu.VMEM_SHARED((NP, D), jnp.float32),
        ],
    )
    def sc_edges(t_hbm, pe_hbm, tg_hbm, z_hbm,
                 out_hbm, pe_v, tg_v, pr, qr, isem, gsem, acc):
        cid = lax.axis_index("c")
        sid = lax.axis_index("s")
        wid = cid * NS + sid
        slab = pl.multiple_of(sid * RPT, 8)
        pltpu.sync_copy(z_hbm, acc.at[pl.ds(slab, RPT)])
        plsc.subcore_barrier()

        def gstart(k, b):
            gc = wid * NCH + k
            i1 = pltpu.async_copy(pe_hbm.at[gc], pe_v.at[b], isem.at[b])
            i2 = pltpu.async_copy(tg_hbm.at[gc], tg_v.at[b], isem.at[b])
            i1.wait()
            i2.wait()
            for ss in range(NS_G):
                sl = pl.ds(ss * SG, SG)
                pltpu.async_copy(
                    t_hbm.at[pe_v.at[b, 0, sl]], pr.at[b, sl], gsem.at[b])
                pltpu.async_copy(
                    t_hbm.at[tg_v.at[b, 0, sl]], qr.at[b, sl], gsem.at[b])

        gstart(0, 0)
        gstart(1, 1)

        def step(k, carry):
            b = lax.bitwise_and(k, 1)
            for ss in range(NS_G):
                sl = pl.ds(ss * SG, SG)
                pltpu.make_async_copy(
                    t_hbm.at[pe_v.at[b, 0, sl]], pr.at[b, sl],
                    gsem.at[b]).wait()
                pltpu.make_async_copy(
                    t_hbm.at[tg_v.at[b, 0, sl]], qr.at[b, sl],
                    gsem.at[b]).wait()

            def erow(e, c2):
                for j in range(8):
                    sl2 = pl.ds(j * 16, 16)
                    xv = pr[b, e, sl2] + qr[b, e, sl2]
                    pr[b, e, sl2] = xv / (1.0 + jnp.exp(-xv))
                return c2

            lax.fori_loop(0, C, erow, 0)
            pltpu.sync_copy(pr.at[b], acc.at[tg_v.at[b, 2]], add=True)

            @pl.when(k + 2 < NCH)
            def _():
                gstart(k + 2, b)

            return carry

        lax.fori_loop(0, NCH, step, 0)
        plsc.subcore_barrier()
        pltpu.sync_copy(acc.at[pl.ds(slab, RPT)],
                        out_hbm.at[cid, pl.ds(slab, RPT)])

    return sc_edges


def _build_sc_counts():
    mesh = plsc.VectorSubcoreMesh(core_axis_name="c", subcore_axis_name="s")

    @functools.partial(
        pl.kernel,
        mesh=mesh,
        out_type=jax.ShapeDtypeStruct((NC, NP, D), jnp.float32),
        scratch_types=[
            pltpu.VMEM((C,), jnp.int32),
            pltpu.VMEM((C, D), jnp.float32),
            pltpu.VMEM_SHARED((NP, D), jnp.float32),
        ],
    )
    def sc_counts(tgt_hbm, z_hbm, out_hbm, tgt_v, g1, acc):
        cid = lax.axis_index("c")
        sid = lax.axis_index("s")
        wid = cid * NS + sid
        slab = pl.multiple_of(sid * RPT, 8)
        pltpu.sync_copy(z_hbm, acc.at[pl.ds(slab, RPT)])
        zv = jnp.zeros((16,), jnp.float32)
        cvec = jnp.where(lax.iota(jnp.int32, 16) == 0,
                         jnp.float32(1.0), jnp.float32(0.0))

        def onerow(e, carry):
            g1[e, pl.ds(0, 16)] = cvec
            for j in range(1, 8):
                g1[e, pl.ds(j * 16, 16)] = zv
            return carry

        lax.fori_loop(0, C, onerow, 0)
        plsc.subcore_barrier()

        def chunk(k, carry):
            base = pl.multiple_of(wid * EPW + k * C, 8)
            pltpu.sync_copy(tgt_hbm.at[pl.ds(base, C)], tgt_v)
            pltpu.sync_copy(g1, acc.at[tgt_v], add=True)
            return carry

        lax.fori_loop(0, EPW // C, chunk, 0)
        plsc.subcore_barrier()
        pltpu.sync_copy(acc.at[pl.ds(slab, RPT)],
                        out_hbm.at[cid, pl.ds(slab, RPT)])

    return sc_counts


_sc_edges = _build_sc_edges()
_sc_counts = _build_sc_counts()


def kernel(x, edge_index, edge_type, credence, edge_emb, W1, b1, W2, b2,
           W_ih, b_ih, W_hh, b_hh):
    src = edge_index[0].astype(jnp.int32)
    tgt = edge_index[1].astype(jnp.int32)
    et = edge_type.astype(jnp.int32)
    cr2 = credence.reshape(N, 1)
    b12 = b1.reshape(1, D)
    b22 = b2.reshape(1, D)
    bih2 = b_ih.reshape(1, 3 * D)
    bhh2 = b_hh.reshape(1, 3 * D)
    zrows = jnp.zeros((RPT, D), jnp.float32)

    nch_tot = E // C
    pe = (5 * src + et).reshape(nch_tot, 1, C)
    tg = jnp.zeros((nch_tot, 4, C), jnp.int32)
    tg = tg.at[:, 0, :].set((5 * tgt + 4).reshape(nch_tot, C))
    tg = tg.at[:, 2, :].set(tgt.reshape(nch_tot, C))

    craw_cnt = _sc_counts(tgt, zrows)
    cnt2 = craw_cnt[:, :, 0:1]

    h = x
    for _ in range(2):
        v = _tc_pre(h, cr2, W1, b12, edge_emb)
        t = v.reshape(5 * N, D)
        sacc = _sc_edges(t, pe, tg, zrows)
        h = _tc_post(sacc, cnt2, h, W2, b22, W_ih, bih2, W_hh, bhh2)
    return h


# speedup vs baseline: 3.7241x; 3.7241x over previous
import functools

import jax
import jax.numpy as jnp
from jax import lax
from jax.experimental import pallas as pl
from jax.experimental.pallas import tpu as pltpu
from jax.experimental.pallas import tpu_sc as plsc

N = 10000
E = 320000
D = 128
NC, NS = 2, 16
NW = NC * NS
EPW = E // NW
C = 80
NP = 10240
RPT = NP // NS
HR = 128
CB = 2000
BN = 2000


def _pre_body(h_ref, cr_ref, w1_ref, b1_ref, ee_ref, v_ref):
    h = h_ref[...]
    p = jnp.dot(h, w1_ref[0:128, :], preferred_element_type=jnp.float32,
                precision=lax.Precision.HIGHEST) + cr_ref[...] * w1_ref[320:321, :]
    q = jnp.dot(h, w1_ref[128:256, :], preferred_element_type=jnp.float32,
                precision=lax.Precision.HIGHEST)
    tb = jnp.dot(ee_ref[...], w1_ref[256:320, :],
                 preferred_element_type=jnp.float32,
                 precision=lax.Precision.HIGHEST) + b1_ref[...]
    parts = [(p + tb[t:t + 1, :])[:, None, :] for t in range(4)]
    parts.append(q[:, None, :])
    v_ref[...] = jnp.concatenate(parts, axis=1)


def _tc_pre(h, cr2, W1, b12, ee):
    return pl.pallas_call(
        _pre_body,
        grid=(N // BN,),
        in_specs=[
            pl.BlockSpec((BN, D), lambda i: (i, 0)),
            pl.BlockSpec((BN, 1), lambda i: (i, 0)),
            pl.BlockSpec((321, D), lambda i: (0, 0)),
            pl.BlockSpec((1, D), lambda i: (0, 0)),
            pl.BlockSpec((4, 64), lambda i: (0, 0)),
        ],
        out_specs=pl.BlockSpec((BN, 5, D), lambda i: (i, 0, 0)),
        out_shape=jax.ShapeDtypeStruct((N, 5, D), jnp.float32),
    )(h, cr2, W1, b12, ee)


def _post_body(s_ref, c_ref, h_ref, w2_ref, b2_ref, wih_ref, bih_ref,
               whh_ref, bhh_ref, out_ref):
    s = s_ref[0] + s_ref[1]
    cnt = jnp.sum(c_ref[...], axis=0)
    h = h_ref[...]
    denom = jnp.maximum(cnt, 1.0)
    sw = jnp.dot(s, w2_ref[...], preferred_element_type=jnp.float32,
                 precision=lax.Precision.HIGHEST)
    agg = (sw + cnt * b2_ref[...]) / denom
    gi = jnp.dot(agg, wih_ref[...], preferred_element_type=jnp.float32,
                 precision=lax.Precision.HIGHEST) + bih_ref[...]
    gh = jnp.dot(h, whh_ref[...], preferred_element_type=jnp.float32,
                 precision=lax.Precision.HIGHEST) + bhh_ref[...]
    rg = jax.nn.sigmoid(gi[:, 0:128] + gh[:, 0:128])
    zg = jax.nn.sigmoid(gi[:, 128:256] + gh[:, 128:256])
    ng = jnp.tanh(gi[:, 256:384] + rg * gh[:, 256:384])
    h_new = (1.0 - zg) * ng + zg * h
    out_ref[...] = jnp.where(cnt > 0, h_new, h)


def _tc_post(sacc, cnt2, h, W2, b22, Wih, bih2, Whh, bhh2):
    return pl.pallas_call(
        _post_body,
        grid=(N // BN,),
        in_specs=[
            pl.BlockSpec((NC, BN, D), lambda i: (0, i, 0)),
            pl.BlockSpec((NC, BN, 1), lambda i: (0, i, 0)),
            pl.BlockSpec((BN, D), lambda i: (i, 0)),
            pl.BlockSpec((D, D), lambda i: (0, 0)),
            pl.BlockSpec((1, D), lambda i: (0, 0)),
            pl.BlockSpec((D, 3 * D), lambda i: (0, 0)),
            pl.BlockSpec((1, 3 * D), lambda i: (0, 0)),
            pl.BlockSpec((D, 3 * D), lambda i: (0, 0)),
            pl.BlockSpec((1, 3 * D), lambda i: (0, 0)),
        ],
        out_specs=pl.BlockSpec((BN, D), lambda i: (i, 0)),
        out_shape=jax.ShapeDtypeStruct((N, D), jnp.float32),
    )(sacc, cnt2, h, W2, b22, Wih, bih2, Whh, bhh2)


def _build_sc_edges():
    mesh = plsc.VectorSubcoreMesh(core_axis_name="c", subcore_axis_name="s")
    NCH = EPW // C
    SG = 16
    NS_G = C // SG

    @functools.partial(
        pl.kernel,
        mesh=mesh,
        out_type=jax.ShapeDtypeStruct((NC, NP, D), jnp.float32),
        scratch_types=[
            pltpu.VMEM((1, C), jnp.int32),
            pltpu.VMEM((1, C), jnp.int32),
            pltpu.VMEM((4, C), jnp.int32),
            pltpu.VMEM((4, C), jnp.int32),
            pltpu.VMEM((C, D), jnp.float32),
            pltpu.VMEM((C, D), jnp.float32),
            pltpu.VMEM((C, D), jnp.float32),
            pltpu.VMEM((C, D), jnp.float32),
            pltpu.SemaphoreType.DMA,
            pltpu.SemaphoreType.DMA,
            pltpu.SemaphoreType.DMA,
            pltpu.SemaphoreType.DMA,
            pltpu.VMEM_SHARED((NP, D), jnp.float32),
        ],
    )
    def sc_edges(t_hbm, pe_hbm, tg_hbm, z_hbm, out_hbm,
                 peA, peB, tgA, tgB, prA, prB, qrA, qrB,
                 isemA, isemB, gsemA, gsemB, acc):
        cid = lax.axis_index("c")
        sid = lax.axis_index("s")
        wid = cid * NS + sid
        slab = pl.multiple_of(sid * RPT, 8)
        pltpu.sync_copy(z_hbm, acc.at[pl.ds(slab, RPT)])
        plsc.subcore_barrier()

        def gstart(k, pe_v, tg_v, pr, qr, isem, gsem):
            gc = wid * NCH + k
            i1 = pltpu.async_copy(pe_hbm.at[gc], pe_v, isem)
            i2 = pltpu.async_copy(tg_hbm.at[gc], tg_v, isem)
            i1.wait()
            i2.wait()
            for ss in range(NS_G):
                sl = pl.ds(ss * SG, SG)
                pltpu.async_copy(t_hbm.at[pe_v.at[0, sl]], pr.at[sl], gsem)
                pltpu.async_copy(t_hbm.at[tg_v.at[0, sl]], qr.at[sl], gsem)

        def process(k, pe_v, tg_v, pr, qr, isem, gsem):
            for ss in range(NS_G):
                sl = pl.ds(ss * SG, SG)
                pltpu.make_async_copy(t_hbm.at[pe_v.at[0, sl]], pr.at[sl],
                                      gsem).wait()
                pltpu.make_async_copy(t_hbm.at[tg_v.at[0, sl]], qr.at[sl],
                                      gsem).wait()

            def erow(e, c2):
                for j in range(8):
                    sl2 = pl.ds(j * 16, 16)
                    xv = pr[e, sl2] + qr[e, sl2]
                    pr[e, sl2] = xv / (1.0 + jnp.exp(-xv))
                return c2

            lax.fori_loop(0, C, erow, 0)
            pltpu.sync_copy(pr, acc.at[tg_v.at[2]], add=True)

            @pl.when(k + 2 < NCH)
            def _():
                gstart(k + 2, pe_v, tg_v, pr, qr, isem, gsem)

        gstart(0, peA, tgA, prA, qrA, isemA, gsemA)
        gstart(1, peB, tgB, prB, qrB, isemB, gsemB)

        def pair(k2, carry):
            process(2 * k2, peA, tgA, prA, qrA, isemA, gsemA)
            process(2 * k2 + 1, peB, tgB, prB, qrB, isemB, gsemB)
            return carry

        lax.fori_loop(0, NCH // 2, pair, 0)
        process(NCH - 1, peA, tgA, prA, qrA, isemA, gsemA)
        plsc.subcore_barrier()
        pltpu.sync_copy(acc.at[pl.ds(slab, RPT)],
                        out_hbm.at[cid, pl.ds(slab, RPT)])

    return sc_edges


def _build_sc_counts():
    mesh = plsc.VectorSubcoreMesh(core_axis_name="c", subcore_axis_name="s")

    @functools.partial(
        pl.kernel,
        mesh=mesh,
        out_type=jax.ShapeDtypeStruct((NC, NP, D), jnp.float32),
        scratch_types=[
            pltpu.VMEM((C,), jnp.int32),
            pltpu.VMEM((C, D), jnp.float32),
            pltpu.VMEM_SHARED((NP, D), jnp.float32),
        ],
    )
    def sc_counts(tgt_hbm, z_hbm, out_hbm, tgt_v, g1, acc):
        cid = lax.axis_index("c")
        sid = lax.axis_index("s")
        wid = cid * NS + sid
        slab = pl.multiple_of(sid * RPT, 8)
        pltpu.sync_copy(z_hbm, acc.at[pl.ds(slab, RPT)])
        zv = jnp.zeros((16,), jnp.float32)
        cvec = jnp.where(lax.iota(jnp.int32, 16) == 0,
                         jnp.float32(1.0), jnp.float32(0.0))

        def onerow(e, carry):
            g1[e, pl.ds(0, 16)] = cvec
            for j in range(1, 8):
                g1[e, pl.ds(j * 16, 16)] = zv
            return carry

        lax.fori_loop(0, C, onerow, 0)
        plsc.subcore_barrier()

        def chunk(k, carry):
            base = pl.multiple_of(wid * EPW + k * C, 8)
            pltpu.sync_copy(tgt_hbm.at[pl.ds(base, C)], tgt_v)
            pltpu.sync_copy(g1, acc.at[tgt_v], add=True)
            return carry

        lax.fori_loop(0, EPW // C, chunk, 0)
        plsc.subcore_barrier()
        pltpu.sync_copy(acc.at[pl.ds(slab, RPT)],
                        out_hbm.at[cid, pl.ds(slab, RPT)])

    return sc_counts


_sc_edges = _build_sc_edges()
_sc_counts = _build_sc_counts()


def kernel(x, edge_index, edge_type, credence, edge_emb, W1, b1, W2, b2,
           W_ih, b_ih, W_hh, b_hh):
    src = edge_index[0].astype(jnp.int32)
    tgt = edge_index[1].astype(jnp.int32)
    et = edge_type.astype(jnp.int32)
    cr2 = credence.reshape(N, 1)
    b12 = b1.reshape(1, D)
    b22 = b2.reshape(1, D)
    bih2 = b_ih.reshape(1, 3 * D)
    bhh2 = b_hh.reshape(1, 3 * D)
    zrows = jnp.zeros((RPT, D), jnp.float32)

    nch_tot = E // C
    pe = (5 * src + et).reshape(nch_tot, 1, C)
    tg = jnp.zeros((nch_tot, 4, C), jnp.int32)
    tg = tg.at[:, 0, :].set((5 * tgt + 4).reshape(nch_tot, C))
    tg = tg.at[:, 2, :].set(tgt.reshape(nch_tot, C))

    craw_cnt = _sc_counts(tgt, zrows)
    cnt2 = craw_cnt[:, :, 0:1]

    h = x
    for _ in range(2):
        v = _tc_pre(h, cr2, W1, b12, edge_emb)
        t = v.reshape(5 * N, D)
        sacc = _sc_edges(t, pe, tg, zrows)
        h = _tc_post(sacc, cnt2, h, W2, b22, W_ih, bih2, W_hh, bhh2)
    return h


# speedup vs baseline: 3.7524x; 1.0076x over previous
import functools

import jax
import jax.numpy as jnp
from jax import lax
from jax.experimental import pallas as pl
from jax.experimental.pallas import tpu as pltpu
from jax.experimental.pallas import tpu_sc as plsc

N = 10000
E = 320000
D = 128
NC, NS = 2, 16
NW = NC * NS
EPW = E // NW
C = 80
NP = 10240
RPT = NP // NS
HR = 128
CB = 2000
BN = 2000


def _pre_body(h_ref, cr_ref, w1_ref, b1_ref, ee_ref, v_ref):
    h = h_ref[...]
    p = jnp.dot(h, w1_ref[0:128, :], preferred_element_type=jnp.float32,
                precision=lax.Precision.HIGHEST) + cr_ref[...] * w1_ref[320:321, :]
    q = jnp.dot(h, w1_ref[128:256, :], preferred_element_type=jnp.float32,
                precision=lax.Precision.HIGHEST)
    tb = jnp.dot(ee_ref[...], w1_ref[256:320, :],
                 preferred_element_type=jnp.float32,
                 precision=lax.Precision.HIGHEST) + b1_ref[...]
    parts = [(p + tb[t:t + 1, :])[:, None, :] for t in range(4)]
    parts.append(q[:, None, :])
    v_ref[...] = jnp.concatenate(parts, axis=1)


def _tc_pre(h, cr2, W1, b12, ee):
    return pl.pallas_call(
        _pre_body,
        grid=(N // BN,),
        in_specs=[
            pl.BlockSpec((BN, D), lambda i: (i, 0)),
            pl.BlockSpec((BN, 1), lambda i: (i, 0)),
            pl.BlockSpec((321, D), lambda i: (0, 0)),
            pl.BlockSpec((1, D), lambda i: (0, 0)),
            pl.BlockSpec((4, 64), lambda i: (0, 0)),
        ],
        out_specs=pl.BlockSpec((BN, 5, D), lambda i: (i, 0, 0)),
        out_shape=jax.ShapeDtypeStruct((N, 5, D), jnp.float32),
    )(h, cr2, W1, b12, ee)


def _post_body(s_ref, c_ref, h_ref, w2_ref, b2_ref, wih_ref, bih_ref,
               whh_ref, bhh_ref, out_ref):
    s = s_ref[0] + s_ref[1]
    cnt = jnp.sum(c_ref[...], axis=0)
    h = h_ref[...]
    denom = jnp.maximum(cnt, 1.0)
    sw = jnp.dot(s, w2_ref[...], preferred_element_type=jnp.float32,
                 precision=lax.Precision.HIGHEST)
    agg = (sw + cnt * b2_ref[...]) / denom
    gi = jnp.dot(agg, wih_ref[...], preferred_element_type=jnp.float32,
                 precision=lax.Precision.HIGHEST) + bih_ref[...]
    gh = jnp.dot(h, whh_ref[...], preferred_element_type=jnp.float32,
                 precision=lax.Precision.HIGHEST) + bhh_ref[...]
    rg = jax.nn.sigmoid(gi[:, 0:128] + gh[:, 0:128])
    zg = jax.nn.sigmoid(gi[:, 128:256] + gh[:, 128:256])
    ng = jnp.tanh(gi[:, 256:384] + rg * gh[:, 256:384])
    h_new = (1.0 - zg) * ng + zg * h
    out_ref[...] = jnp.where(cnt > 0, h_new, h)


def _tc_post(sacc, cnt2, h, W2, b22, Wih, bih2, Whh, bhh2):
    return pl.pallas_call(
        _post_body,
        grid=(N // BN,),
        in_specs=[
            pl.BlockSpec((NC, BN, D), lambda i: (0, i, 0)),
            pl.BlockSpec((NC, BN, 1), lambda i: (0, i, 0)),
            pl.BlockSpec((BN, D), lambda i: (i, 0)),
            pl.BlockSpec((D, D), lambda i: (0, 0)),
            pl.BlockSpec((1, D), lambda i: (0, 0)),
            pl.BlockSpec((D, 3 * D), lambda i: (0, 0)),
            pl.BlockSpec((1, 3 * D), lambda i: (0, 0)),
            pl.BlockSpec((D, 3 * D), lambda i: (0, 0)),
            pl.BlockSpec((1, 3 * D), lambda i: (0, 0)),
        ],
        out_specs=pl.BlockSpec((BN, D), lambda i: (i, 0)),
        out_shape=jax.ShapeDtypeStruct((N, D), jnp.float32),
    )(sacc, cnt2, h, W2, b22, Wih, bih2, Whh, bhh2)


def _build_sc_edges():
    mesh = plsc.VectorSubcoreMesh(core_axis_name="c", subcore_axis_name="s")
    NCH = EPW // C
    SG = 80
    NS_G = C // SG

    @functools.partial(
        pl.kernel,
        mesh=mesh,
        out_type=jax.ShapeDtypeStruct((NC, NP, D), jnp.float32),
        scratch_types=[
            pltpu.VMEM((1, C), jnp.int32),
            pltpu.VMEM((1, C), jnp.int32),
            pltpu.VMEM((4, C), jnp.int32),
            pltpu.VMEM((4, C), jnp.int32),
            pltpu.VMEM((C, D), jnp.float32),
            pltpu.VMEM((C, D), jnp.float32),
            pltpu.VMEM((C, D), jnp.float32),
            pltpu.VMEM((C, D), jnp.float32),
            pltpu.SemaphoreType.DMA,
            pltpu.SemaphoreType.DMA,
            pltpu.SemaphoreType.DMA,
            pltpu.SemaphoreType.DMA,
            pltpu.VMEM_SHARED((NP, D), jnp.float32),
        ],
    )
    def sc_edges(t_hbm, pe_hbm, tg_hbm, z_hbm, out_hbm,
                 peA, peB, tgA, tgB, prA, prB, qrA, qrB,
                 isemA, isemB, gsemA, gsemB, acc):
        cid = lax.axis_index("c")
        sid = lax.axis_index("s")
        wid = cid * NS + sid
        slab = pl.multiple_of(sid * RPT, 8)
        pltpu.sync_copy(z_hbm, acc.at[pl.ds(slab, RPT)])
        plsc.subcore_barrier()

        def gstart(k, pe_v, tg_v, pr, qr, isem, gsem):
            gc = wid * NCH + k
            i1 = pltpu.async_copy(pe_hbm.at[gc], pe_v, isem)
            i2 = pltpu.async_copy(tg_hbm.at[gc], tg_v, isem)
            i1.wait()
            i2.wait()
            for ss in range(NS_G):
                sl = pl.ds(ss * SG, SG)
                pltpu.async_copy(t_hbm.at[pe_v.at[0, sl]], pr.at[sl], gsem)
                pltpu.async_copy(t_hbm.at[tg_v.at[0, sl]], qr.at[sl], gsem)

        def process(k, pe_v, tg_v, pr, qr, isem, gsem):
            for ss in range(NS_G):
                sl = pl.ds(ss * SG, SG)
                pltpu.make_async_copy(t_hbm.at[pe_v.at[0, sl]], pr.at[sl],
                                      gsem).wait()
                pltpu.make_async_copy(t_hbm.at[tg_v.at[0, sl]], qr.at[sl],
                                      gsem).wait()

            def erow(e, c2):
                for j in range(8):
                    sl2 = pl.ds(j * 16, 16)
                    xv = pr[e, sl2] + qr[e, sl2]
                    pr[e, sl2] = xv / (1.0 + jnp.exp(-xv))
                return c2

            lax.fori_loop(0, C, erow, 0)
            pltpu.sync_copy(pr, acc.at[tg_v.at[2]], add=True)

            @pl.when(k + 2 < NCH)
            def _():
                gstart(k + 2, pe_v, tg_v, pr, qr, isem, gsem)

        gstart(0, peA, tgA, prA, qrA, isemA, gsemA)
        gstart(1, peB, tgB, prB, qrB, isemB, gsemB)

        def pair(k2, carry):
            process(2 * k2, peA, tgA, prA, qrA, isemA, gsemA)
            process(2 * k2 + 1, peB, tgB, prB, qrB, isemB, gsemB)
            return carry

        lax.fori_loop(0, NCH // 2, pair, 0)
        process(NCH - 1, peA, tgA, prA, qrA, isemA, gsemA)
        plsc.subcore_barrier()
        pltpu.sync_copy(acc.at[pl.ds(slab, RPT)],
                        out_hbm.at[cid, pl.ds(slab, RPT)])

    return sc_edges


def _build_sc_counts():
    mesh = plsc.VectorSubcoreMesh(core_axis_name="c", subcore_axis_name="s")

    @functools.partial(
        pl.kernel,
        mesh=mesh,
        out_type=jax.ShapeDtypeStruct((NC, NP, D), jnp.float32),
        scratch_types=[
            pltpu.VMEM((C,), jnp.int32),
            pltpu.VMEM((C, D), jnp.float32),
            pltpu.VMEM_SHARED((NP, D), jnp.float32),
        ],
    )
    def sc_counts(tgt_hbm, z_hbm, out_hbm, tgt_v, g1, acc):
        cid = lax.axis_index("c")
        sid = lax.axis_index("s")
        wid = cid * NS + sid
        slab = pl.multiple_of(sid * RPT, 8)
        pltpu.sync_copy(z_hbm, acc.at[pl.ds(slab, RPT)])
        zv = jnp.zeros((16,), jnp.float32)
        cvec = jnp.where(lax.iota(jnp.int32, 16) == 0,
                         jnp.float32(1.0), jnp.float32(0.0))

        def onerow(e, carry):
            g1[e, pl.ds(0, 16)] = cvec
            for j in range(1, 8):
                g1[e, pl.ds(j * 16, 16)] = zv
            return carry

        lax.fori_loop(0, C, onerow, 0)
        plsc.subcore_barrier()

        def chunk(k, carry):
            base = pl.multiple_of(wid * EPW + k * C, 8)
            pltpu.sync_copy(tgt_hbm.at[pl.ds(base, C)], tgt_v)
            pltpu.sync_copy(g1, acc.at[tgt_v], add=True)
            return carry

        lax.fori_loop(0, EPW // C, chunk, 0)
        plsc.subcore_barrier()
        pltpu.sync_copy(acc.at[pl.ds(slab, RPT)],
                        out_hbm.at[cid, pl.ds(slab, RPT)])

    return sc_counts


_sc_edges = _build_sc_edges()
_sc_counts = _build_sc_counts()


def kernel(x, edge_index, edge_type, credence, edge_emb, W1, b1, W2, b2,
           W_ih, b_ih, W_hh, b_hh):
    src = edge_index[0].astype(jnp.int32)
    tgt = edge_index[1].astype(jnp.int32)
    et = edge_type.astype(jnp.int32)
    cr2 = credence.reshape(N, 1)
    b12 = b1.reshape(1, D)
    b22 = b2.reshape(1, D)
    bih2 = b_ih.reshape(1, 3 * D)
    bhh2 = b_hh.reshape(1, 3 * D)
    zrows = jnp.zeros((RPT, D), jnp.float32)

    nch_tot = E // C
    pe = (5 * src + et).reshape(nch_tot, 1, C)
    tg = jnp.zeros((nch_tot, 4, C), jnp.int32)
    tg = tg.at[:, 0, :].set((5 * tgt + 4).reshape(nch_tot, C))
    tg = tg.at[:, 2, :].set(tgt.reshape(nch_tot, C))

    craw_cnt = _sc_counts(tgt, zrows)
    cnt2 = craw_cnt[:, :, 0:1]

    h = x
    for _ in range(2):
        v = _tc_pre(h, cr2, W1, b12, edge_emb)
        t = v.reshape(5 * N, D)
        sacc = _sc_edges(t, pe, tg, zrows)
        h = _tc_post(sacc, cnt2, h, W2, b22, W_ih, bih2, W_hh, bhh2)
    return h
